# baseline (speedup 1.0000x reference)
import jax
import jax.numpy as jnp
from jax import lax
from jax.experimental import pallas as pl
from jax.experimental.pallas import tpu as pltpu
from jax.experimental.pallas import tpu_sc as plsc

_B = 16384
_D = 128
_NC = 2
_NS = 16
_NW = _NC * _NS
_BPW = _B // _NW
_NCH = 2
_CH = _BPW // _NCH


def _sc_body(idx_hbm, cond_hbm, table_hbm, out_hbm, idx_v, table_v, buf_v,
             *sems):
    sem_t, sem_i = sems[0], sems[1]
    sem_in = list(sems[2:2 + _NCH])
    sem_out = list(sems[2 + _NCH:2 + 2 * _NCH])
    wid = lax.axis_index("s") * _NC + lax.axis_index("c")
    base = wid * _BPW

    h_t = pltpu.async_copy(table_hbm, table_v, sem_t)
    h_i = pltpu.async_copy(idx_hbm.at[pl.ds(base, _BPW)], idx_v, sem_i)
    h_in = [
        pltpu.async_copy(
            cond_hbm.at[pl.ds((base + c * _CH) * _D, _CH * _D)],
            buf_v.at[pl.ds(c * _CH * _D, _CH * _D)],
            sem_in[c],
        )
        for c in range(_NCH)
    ]
    h_t.wait()
    h_i.wait()

    t1 = [table_v[pl.ds(_D + 16 * j, 16)] for j in range(8)]
    d = [table_v[pl.ds(16 * j, 16)] - t1[j] for j in range(8)]

    h_out = []
    for c in range(_NCH):
        h_in[c].wait()

        @plsc.parallel_loop(0, _CH // 16, unroll=3)
        def group_body(g, _c=c):
            gb = _c * _CH + g * 16
            fv = 1.0 - idx_v[pl.ds(gb, 16)].astype(jnp.float32)
            for r in range(16):
                f = lax.broadcast_in_dim(lax.slice(fv, (r,), (r + 1,)), (16,), (0,))
                rb = (gb + r) * _D
                for j in range(8):
                    off = pl.ds(rb + 16 * j, 16)
                    plsc.addupdate(buf_v.at[off], t1[j] + f * d[j])

        h_out.append(
            pltpu.async_copy(
                buf_v.at[pl.ds(c * _CH * _D, _CH * _D)],
                out_hbm.at[pl.ds((base + c * _CH) * _D, _CH * _D)],
                sem_out[c],
            )
        )
    for h in h_out:
        h.wait()


@jax.jit
def _run(idx, cond_flat, table_flat):
    mesh = plsc.VectorSubcoreMesh(core_axis_name="c", subcore_axis_name="s")
    return pl.kernel(
        _sc_body,
        out_type=jax.ShapeDtypeStruct((_B * _D,), jnp.float32),
        mesh=mesh,
        scratch_types=[
            pltpu.VMEM((_BPW,), jnp.int32),
            pltpu.VMEM((2 * _D,), jnp.float32),
            pltpu.VMEM((_BPW * _D,), jnp.float32),
        ] + [pltpu.SemaphoreType.DMA] * (2 + 2 * _NCH),
    )(idx, cond_flat, table_flat)


def kernel(indices, cond, table):
    idx = indices.astype(jnp.int32)
    out_flat = _run(idx, cond.reshape(-1), table.reshape(-1))
    return out_flat.reshape(_B, _D)


# speedup vs baseline: 1.0470x; 1.0470x over previous
import jax
import jax.numpy as jnp
from jax import lax
from jax.experimental import pallas as pl
from jax.experimental.pallas import tpu as pltpu
from jax.experimental.pallas import tpu_sc as plsc

_B = 16384
_D = 128
_NC = 2
_NS = 16
_NW = _NC * _NS
_BPW = _B // _NW
_NCH = 2
_CH = _BPW // _NCH


def _sc_body(idx_hbm, cond_hbm, table_hbm, out_hbm, idx_v, table_v, buf_v,
             *sems):
    sem_t, sem_i = sems[0], sems[1]
    sem_in = list(sems[2:2 + _NCH])
    sem_out = list(sems[2 + _NCH:2 + 2 * _NCH])
    wid = lax.axis_index("s") * _NC + lax.axis_index("c")
    base = wid * _BPW

    h_t = pltpu.async_copy(table_hbm, table_v, sem_t)
    h_i = pltpu.async_copy(idx_hbm.at[pl.ds(base, _BPW)], idx_v, sem_i)
    h_in = [
        pltpu.async_copy(
            cond_hbm.at[pl.ds((base + c * _CH) * _D, _CH * _D)],
            buf_v.at[pl.ds(c * _CH * _D, _CH * _D)],
            sem_in[c],
        )
        for c in range(_NCH)
    ]
    h_t.wait()
    h_i.wait()

    t1 = [table_v[pl.ds(_D + 16 * j, 16)] for j in range(8)]
    d = [table_v[pl.ds(16 * j, 16)] - t1[j] for j in range(8)]

    h_out = []
    for c in range(_NCH):
        h_in[c].wait()

        @plsc.parallel_loop(0, _CH // 16, unroll=2)
        def group_body(g, _c=c):
            gb = _c * _CH + g * 16
            fv = 1.0 - idx_v[pl.ds(gb, 16)].astype(jnp.float32)
            for r in range(16):
                f = lax.broadcast_in_dim(lax.slice(fv, (r,), (r + 1,)), (16,), (0,))
                rb = (gb + r) * _D
                for j in range(8):
                    off = pl.ds(rb + 16 * j, 16)
                    plsc.addupdate(buf_v.at[off], t1[j] + f * d[j])

        h_out.append(
            pltpu.async_copy(
                buf_v.at[pl.ds(c * _CH * _D, _CH * _D)],
                out_hbm.at[pl.ds((base + c * _CH) * _D, _CH * _D)],
                sem_out[c],
            )
        )
    for h in h_out:
        h.wait()


@jax.jit
def _run(idx, cond_flat, table_flat):
    mesh = plsc.VectorSubcoreMesh(core_axis_name="c", subcore_axis_name="s")
    return pl.kernel(
        _sc_body,
        out_type=jax.ShapeDtypeStruct((_B * _D,), jnp.float32),
        mesh=mesh,
        scratch_types=[
            pltpu.VMEM((_BPW,), jnp.int32),
            pltpu.VMEM((2 * _D,), jnp.float32),
            pltpu.VMEM((_BPW * _D,), jnp.float32),
        ] + [pltpu.SemaphoreType.DMA] * (2 + 2 * _NCH),
    )(idx, cond_flat, table_flat)


def kernel(indices, cond, table):
    idx = indices.astype(jnp.int32)
    out_flat = _run(idx, cond.reshape(-1), table.reshape(-1))
    return out_flat.reshape(_B, _D)
